# baseline (speedup 1.0000x reference)
import jax
import jax.numpy as jnp
from jax import lax
from jax.experimental import pallas as pl
from jax.experimental.pallas import tpu as pltpu
from jax.experimental.pallas import tpu_sc as plsc

N = 32768
H = 16384
CP = 80
NC, NS = 2, 16
NW = NC * NS

_MESH = dict(core_axis_name="c", subcore_axis_name="s", num_cores=NC,
             num_subcores=NS)


def _leaky(x):
    return jnp.where(x >= 0, x, 0.1 * x)


def _dot(a, b):
    return jnp.dot(a, b, preferred_element_type=jnp.float32)


_BN1 = 1024


def _points_body(pc1T, el1T, ba1T, pc2T, el2T, ba2T, W1T, b1, W2T, b2, W3T,
                 b3, o1, o2):
    zpad = jnp.zeros((_BN1, 60), jnp.float32)
    for pcT, elT, baT, o in ((pc1T, el1T, ba1T, o1), (pc2T, el2T, ba2T, o2)):
        h = _leaky(_dot(pcT[...], W1T[...]) + b1[...])
        h = _leaky(_dot(h, W2T[...]) + b2[...])
        f = _leaky(_dot(h, W3T[...]) + b3[...])
        feat = jnp.concatenate([elT[...], f, zpad], axis=1)
        ba = baT[...]
        for d in range(4):
            o[d] = feat * ba[:, d:d + 1]


def _stage_points(pc1T, el1T, ba1T, pc2T, el2T, ba2T, W1T, b1r, W2T, b2r,
                  W3T, b3r):
    bn = lambda c: pl.BlockSpec((_BN1, c), lambda i: (i, 0))
    wf = lambda s: pl.BlockSpec(s, lambda i: (0,) * len(s))
    return pl.pallas_call(
        _points_body,
        grid=(N // _BN1,),
        in_specs=[bn(3), bn(4), bn(4), bn(3), bn(4), bn(4),
                  wf((3, 32)), wf((1, 32)), wf((32, 32)), wf((1, 32)),
                  wf((32, 64)), wf((1, 64))],
        out_specs=[pl.BlockSpec((4, _BN1, 128), lambda i: (0, i, 0))] * 2,
        out_shape=[jax.ShapeDtypeStruct((4, N, 128), jnp.float32)] * 2,
    )(pc1T, el1T, ba1T, pc2T, el2T, ba2T, W1T, b1r, W2T, b2r, W3T, b3r)


_SPC = 256
_RPT = 4 * N // NS


_BGT = H // NS


def _splat_blur_body(w1, i1, w2, i2, zrows, n1, n2, g1, g2, shared, buf,
                     sidx, nidx):
    cid = lax.axis_index("c")
    sid = lax.axis_index("s")
    zsl = H // NS
    zoff = pl.multiple_of(sid * zsl, zsl)
    for z in range(zsl // _SPC):
        pltpu.sync_copy(zrows, buf)
        pltpu.sync_copy(buf, shared.at[pl.ds(zoff + z * _SPC, _SPC)])
    plsc.subcore_barrier()

    def scatter(whbm, ihbm):
        base = sid * _RPT
        for c in range(_RPT // _SPC):
            if c % 16 == 0:
                pltpu.sync_copy(ihbm.at[sid, pl.ds((c // 16) * 32, 32)], sidx)
            off = pl.multiple_of(base + c * _SPC, _SPC)
            pltpu.sync_copy(whbm.at[pl.ds(off, _SPC), pl.ds(0, CP)], buf)
            for j in range(_SPC // 128):
                pltpu.sync_copy(buf.at[pl.ds(j * 128, 128)],
                                shared.at[sidx.at[2 * (c % 16) + j]],
                                add=True)

    @pl.when(cid == 0)
    def _():
        scatter(w1, i1)

    @pl.when(cid == 1)
    def _():
        scatter(w2, i2)

    plsc.subcore_barrier()

    def gather(nf, g):
        pltpu.sync_copy(nf.at[sid], nidx)
        gbase = pl.multiple_of(sid * _BGT, _BGT)
        for k in range(9):
            for q in range(_BGT // 256):
                for j in range(2):
                    r = k * (_BGT // 128) + q * 2 + j
                    pltpu.sync_copy(shared.at[nidx.at[r]],
                                    buf.at[pl.ds(j * 128, 128)])
                pltpu.sync_copy(buf, g.at[k, pl.ds(gbase + q * 256, 256),
                                          pl.ds(0, CP)])

    @pl.when(cid == 0)
    def _():
        gather(n1, g1)

    @pl.when(cid == 1)
    def _():
        gather(n2, g2)


def _stage_splat_blur(w1rows, i1, w2rows, i2, nbr1, nbr2):
    zrows = jnp.zeros((_SPC, CP), jnp.float32)
    f = pl.kernel(
        _splat_blur_body,
        out_type=[jax.ShapeDtypeStruct((9, H, 128), jnp.float32)] * 2,
        mesh=plsc.VectorSubcoreMesh(**_MESH),
        compiler_params=pltpu.CompilerParams(use_tc_tiling_on_sc=False),
        scratch_types=[pltpu.VMEM_SHARED((H, CP), jnp.float32),
                       pltpu.VMEM((_SPC, CP), jnp.float32),
                       pltpu.VMEM((32, 128), jnp.int32),
                       pltpu.VMEM((9 * _BGT // 128, 128), jnp.int32)],
    )
    return f(w1rows, i1, w2rows, i2, zrows, nbr1, nbr2)


_BH3 = 512


def _blur_conv_body(g1, g2, wbk, bbr, o1, o2):
    for g, o in ((g1, o1), (g2, o2)):
        s = jnp.zeros((_BH3, 64), jnp.float32)
        for k in range(9):
            s = s + _dot(g[k][:, :CP], wbk[k])
        o[...] = _leaky(s + bbr[...])


def _stage_blur_conv(g1, g2, wbk, bbr):
    gs = pl.BlockSpec((9, _BH3, 128), lambda i: (0, i, 0))
    return pl.pallas_call(
        _blur_conv_body,
        grid=(H // _BH3,),
        in_specs=[gs, gs, pl.BlockSpec((9, CP, 64), lambda i: (0, 0, 0)),
                  pl.BlockSpec((1, 64), lambda i: (0, 0))],
        out_specs=[pl.BlockSpec((_BH3, 64), lambda i: (i, 0))] * 2,
        out_shape=[jax.ShapeDtypeStruct((H, 64), jnp.float32)] * 2,
    )(g1, g2, wbk, bbr)


_CGC = H // NW


def _corr_gather_body(l1, c1, l2, c2, p, bufA, bufB, idx1, idx2):
    cid = lax.axis_index("c")
    sid = lax.axis_index("s")
    wid = cid * NS + sid
    base = pl.multiple_of(wid * _CGC, _CGC)
    pltpu.sync_copy(c1.at[wid], idx1)
    pltpu.sync_copy(c2.at[wid], idx2)

    def mul_row(r, carry):
        for cc in range(4):
            sl = pl.ds(cc * 16, 16)
            bufA[r, sl] = bufA[r, sl] * bufB[r, sl]
        return carry

    for k in range(27):
        for hh in range(_CGC // 256):
            for j in range(2):
                r = k * (_CGC // 128) + hh * 2 + j
                pltpu.sync_copy(l1.at[idx1.at[r]],
                                bufA.at[pl.ds(j * 128, 128)])
                pltpu.sync_copy(l2.at[idx2.at[r]],
                                bufB.at[pl.ds(j * 128, 128)])
            lax.fori_loop(0, 256, mul_row, 0)
            pltpu.sync_copy(bufA, p.at[k // 2, pl.ds(base + hh * 256, 256),
                                       pl.ds(64 * (k % 2), 64)])


def _stage_corr_gather(lat1, co1, lat2, co2):
    f = pl.kernel(
        _corr_gather_body,
        out_type=jax.ShapeDtypeStruct((14, H, 128), jnp.float32),
        mesh=plsc.VectorSubcoreMesh(**_MESH),
        compiler_params=pltpu.CompilerParams(use_tc_tiling_on_sc=False),
        scratch_types=[pltpu.VMEM((256, 64), jnp.float32),
                       pltpu.VMEM((256, 64), jnp.float32),
                       pltpu.VMEM((112, 128), jnp.int32),
                       pltpu.VMEM((112, 128), jnp.int32)],
    )
    return f(lat1, co1, lat2, co2)


_BH4 = 512


def _corr_conv_body(p, wc1t, bc1r, wc2t, bc2r, o):
    s = jnp.zeros((_BH4, 32), jnp.float32)
    for k in range(27):
        blk = p[k // 2][:, 64 * (k % 2):64 * (k % 2) + 64]
        s = s + _leaky(_dot(blk, wc1t[...]) + bc1r[...])
    m = s * (1.0 / 27.0)
    o[...] = _leaky(_dot(m, wc2t[...]) + bc2r[...])


def _stage_corr_conv(p, wc1t, bc1r, wc2t, bc2r):
    return pl.pallas_call(
        _corr_conv_body,
        grid=(H // _BH4,),
        in_specs=[pl.BlockSpec((14, _BH4, 128), lambda i: (0, i, 0)),
                  pl.BlockSpec((64, 32), lambda i: (0, 0)),
                  pl.BlockSpec((1, 32), lambda i: (0, 0)),
                  pl.BlockSpec((32, 32), lambda i: (0, 0)),
                  pl.BlockSpec((1, 32), lambda i: (0, 0))],
        out_specs=pl.BlockSpec((_BH4, 32), lambda i: (i, 0)),
        out_shape=jax.ShapeDtypeStruct((H, 32), jnp.float32),
    )(p, wc1t, bc1r, wc2t, bc2r)


_SLC = N // NW


def _slice_gather_body(hlat, lat1, offf, gh, gu, bufh, bufu, idxbuf):
    cid = lax.axis_index("c")
    sid = lax.axis_index("s")
    wid = cid * NS + sid
    base = pl.multiple_of(wid * _SLC, _SLC)
    pltpu.sync_copy(offf.at[wid], idxbuf)
    for d in range(4):
        for q in range(_SLC // 256):
            for j in range(2):
                r = d * (_SLC // 128) + q * 2 + j
                pltpu.sync_copy(hlat.at[idxbuf.at[r]],
                                bufh.at[pl.ds(j * 128, 128)])
                pltpu.sync_copy(lat1.at[idxbuf.at[r]],
                                bufu.at[pl.ds(j * 128, 128)])
            pltpu.sync_copy(bufh, gh.at[pl.ds(base + q * 256, 256),
                                        pl.ds(d * 32, 32)])
            pltpu.sync_copy(bufu, gu.at[pl.ds(base + q * 256, 256),
                                        pl.ds(d * 64, 64)])


def _stage_slice_gather(hlat, lat1, offf):
    f = pl.kernel(
        _slice_gather_body,
        out_type=[jax.ShapeDtypeStruct((N, 128), jnp.float32),
                  jax.ShapeDtypeStruct((N, 256), jnp.float32)],
        mesh=plsc.VectorSubcoreMesh(**_MESH),
        compiler_params=pltpu.CompilerParams(use_tc_tiling_on_sc=False),
        scratch_types=[pltpu.VMEM((256, 32), jnp.float32),
                       pltpu.VMEM((256, 64), jnp.float32),
                       pltpu.VMEM((4 * _SLC // 128, 128), jnp.int32)],
    )
    return f(hlat, lat1, offf)


_BN5 = 512


def _final_body(gh, gu, baT, elT, wr1e, wr1c, br1r, wr2t, br2r, wr3t, br3r,
                wf2u, wf2r, bf2r, wf3t, bf3r, wf4t, bf4r, o):
    ba = baT[...]
    ghv, guv = gh[...], gu[...]
    corr = jnp.zeros((_BN5, 32), jnp.float32)
    up = jnp.zeros((_BN5, 64), jnp.float32)
    for d in range(4):
        w = ba[:, d:d + 1]
        corr = corr + ghv[:, d * 32:(d + 1) * 32] * w
        up = up + guv[:, d * 64:(d + 1) * 64] * w
    bf = jnp.bfloat16
    r = _leaky(_dot(elT[...], wr1e[...]) + _dot(corr, wr1c[...]) + br1r[...])
    r = _leaky(_dot(r, wr2t[...]) + br2r[...])
    r = _leaky(_dot(r, wr3t[...]) + br3r[...])
    x = _leaky(_dot(up.astype(bf), wf2u[...].astype(bf))
               + _dot(r.astype(bf), wf2r[...].astype(bf)) + bf2r[...])
    x = _leaky(_dot(x.astype(bf), wf3t[...].astype(bf)) + bf3r[...])
    o[...] = _dot(x.astype(bf), wf4t[...].astype(bf)) + bf4r[...]


def _stage_final(gh, gu, baT, elT, weights):
    bs = lambda *s: pl.BlockSpec(s, lambda i: (0,) * len(s))
    return pl.pallas_call(
        _final_body,
        grid=(N // _BN5,),
        in_specs=[pl.BlockSpec((_BN5, 128), lambda i: (i, 0)),
                  pl.BlockSpec((_BN5, 256), lambda i: (i, 0)),
                  pl.BlockSpec((_BN5, 4), lambda i: (i, 0)),
                  pl.BlockSpec((_BN5, 4), lambda i: (i, 0)),
                  bs(4, 64), bs(32, 64), bs(1, 64),
                  bs(64, 64), bs(1, 64), bs(64, 64), bs(1, 64),
                  bs(64, 1024), bs(64, 1024), bs(1, 1024),
                  bs(1024, 512), bs(1, 512), bs(512, 128), bs(1, 128)],
        out_specs=pl.BlockSpec((_BN5, 128), lambda i: (i, 0)),
        out_shape=jax.ShapeDtypeStruct((N, 128), jnp.float32),
    )(gh, gu, baT, elT, *weights)


def kernel(pc1, pc2, pc1_el_minus_gr, pc2_el_minus_gr, pc1_barycentric,
           pc2_barycentric, pc1_lattice_offset, pc2_lattice_offset,
           pc1_blur_neighbors, pc2_blur_neighbors, pc1_corr_indices,
           pc2_corr_indices, W1, b1, W2, b2, W3, b3, Wb, bb, Wc1, bc1, Wc2,
           bc2, Wr1, br1, Wr2, br2, Wr3, br3, Wf2, bf2, Wf3, bf3, Wf4, bf4):
    i32 = jnp.int32
    pc1T, pc2T = pc1[0].T, pc2[0].T
    el1T, el2T = pc1_el_minus_gr[0].T, pc2_el_minus_gr[0].T
    ba1T, ba2T = pc1_barycentric[0].T, pc2_barycentric[0].T
    def wmaj(a, k, c):
        r = a.astype(i32).reshape(k, NW, c).transpose(1, 0, 2)
        r = r.reshape(NW, k * c // 128, 128)
        pad = (-r.shape[1]) % 8
        return jnp.pad(r, ((0, 0), (0, pad), (0, 0)))
    idx1 = pc1_lattice_offset[0].astype(i32).reshape(NS, _RPT // 128, 128)
    idx2 = pc2_lattice_offset[0].astype(i32).reshape(NS, _RPT // 128, 128)
    tmaj = lambda a, k, c: (a.astype(i32).reshape(k, NS, c)
                            .transpose(1, 0, 2).reshape(NS, k * c // 128, 128))
    nbr1 = tmaj(pc1_blur_neighbors[0], 9, _BGT)
    nbr2 = tmaj(pc2_blur_neighbors[0], 9, _BGT)
    co1 = wmaj(pc1_corr_indices[0], 27, _CGC)
    co2 = wmaj(pc2_corr_indices[0], 27, _CGC)
    off1 = wmaj(pc1_lattice_offset[0], 4, _SLC)

    r1 = lambda b: b.reshape(1, -1)
    wbk = jnp.pad(jnp.transpose(Wb.reshape(64, 68, 9), (2, 1, 0)),
                  ((0, 0), (0, 12), (0, 0)))
    wf4t = jnp.pad(Wf4.T, ((0, 0), (0, 125)))
    bf4p = jnp.pad(bf4, (0, 125)).reshape(1, 128)

    w1r, w2r = _stage_points(pc1T, el1T, ba1T, pc2T, el2T, ba2T,
                             W1.T, r1(b1), W2.T, r1(b2), W3.T, r1(b3))
    g1, g2 = _stage_splat_blur(w1r.reshape(4 * N, 128), idx1,
                               w2r.reshape(4 * N, 128), idx2, nbr1, nbr2)
    lat1, lat2 = _stage_blur_conv(g1, g2, wbk, r1(bb))
    p = _stage_corr_gather(lat1, co1, lat2, co2)
    hlat = _stage_corr_conv(p, Wc1.T, r1(bc1), Wc2.T, r1(bc2))
    gh, gu = _stage_slice_gather(hlat, lat1, off1)
    weights = (Wr1[:, :4].T, Wr1[:, 4:].T, r1(br1), Wr2.T, r1(br2), Wr3.T,
               r1(br3), Wf2[:, :64].T, Wf2[:, 64:].T, r1(bf2), Wf3.T,
               r1(bf3), wf4t, bf4p)
    o = _stage_final(gh, gu, ba1T, el1T, weights)
    return o[:, :3].T[None]


# speedup vs baseline: 1.0383x; 1.0383x over previous
import jax
import jax.numpy as jnp
from jax import lax
from jax.experimental import pallas as pl
from jax.experimental.pallas import tpu as pltpu
from jax.experimental.pallas import tpu_sc as plsc

N = 32768
H = 16384
CP = 80
NC, NS = 2, 16
NW = NC * NS

_MESH = dict(core_axis_name="c", subcore_axis_name="s", num_cores=NC,
             num_subcores=NS)


def _leaky(x):
    return jnp.where(x >= 0, x, 0.1 * x)


def _dot(a, b):
    return jnp.dot(a, b, preferred_element_type=jnp.float32)


_BN1 = 1024


def _points_body(pc1T, el1T, ba1T, pc2T, el2T, ba2T, W1T, b1, W2T, b2, W3T,
                 b3, o1, o2):
    zpad = jnp.zeros((_BN1, 60), jnp.float32)
    for pcT, elT, baT, o in ((pc1T, el1T, ba1T, o1), (pc2T, el2T, ba2T, o2)):
        h = _leaky(_dot(pcT[...], W1T[...]) + b1[...])
        h = _leaky(_dot(h, W2T[...]) + b2[...])
        f = _leaky(_dot(h, W3T[...]) + b3[...])
        feat = jnp.concatenate([elT[...], f, zpad], axis=1)
        ba = baT[...]
        for d in range(4):
            o[d] = feat * ba[:, d:d + 1]


def _stage_points(pc1T, el1T, ba1T, pc2T, el2T, ba2T, W1T, b1r, W2T, b2r,
                  W3T, b3r):
    bn = lambda c: pl.BlockSpec((_BN1, c), lambda i: (i, 0))
    wf = lambda s: pl.BlockSpec(s, lambda i: (0,) * len(s))
    return pl.pallas_call(
        _points_body,
        grid=(N // _BN1,),
        in_specs=[bn(3), bn(4), bn(4), bn(3), bn(4), bn(4),
                  wf((3, 32)), wf((1, 32)), wf((32, 32)), wf((1, 32)),
                  wf((32, 64)), wf((1, 64))],
        out_specs=[pl.BlockSpec((4, _BN1, 128), lambda i: (0, i, 0))] * 2,
        out_shape=[jax.ShapeDtypeStruct((4, N, 128), jnp.float32)] * 2,
    )(pc1T, el1T, ba1T, pc2T, el2T, ba2T, W1T, b1r, W2T, b2r, W3T, b3r)


_SPC = 256
_RPT = 4 * N // NS


_BGT = H // NS


def _splat_blur_body(w1, i1, w2, i2, zrows, n1, n2, g1, g2, shared, buf,
                     sidx, nidx):
    cid = lax.axis_index("c")
    sid = lax.axis_index("s")
    zsl = H // NS
    zoff = pl.multiple_of(sid * zsl, zsl)
    for z in range(zsl // _SPC):
        pltpu.sync_copy(zrows, buf)
        pltpu.sync_copy(buf, shared.at[pl.ds(zoff + z * _SPC, _SPC)])
    plsc.subcore_barrier()

    def scatter(whbm, ihbm):
        base = sid * _RPT
        for c in range(_RPT // _SPC):
            if c % 16 == 0:
                pltpu.sync_copy(ihbm.at[sid, pl.ds((c // 16) * 32, 32)], sidx)
            off = pl.multiple_of(base + c * _SPC, _SPC)
            pltpu.sync_copy(whbm.at[pl.ds(off, _SPC), pl.ds(0, CP)], buf)
            for j in range(_SPC // 128):
                pltpu.sync_copy(buf.at[pl.ds(j * 128, 128)],
                                shared.at[sidx.at[2 * (c % 16) + j]],
                                add=True)

    @pl.when(cid == 0)
    def _():
        scatter(w1, i1)

    @pl.when(cid == 1)
    def _():
        scatter(w2, i2)

    plsc.subcore_barrier()

    def gather(nf, g):
        pltpu.sync_copy(nf.at[sid], nidx)
        gbase = pl.multiple_of(sid * _BGT, _BGT)
        for k in range(9):
            for q in range(_BGT // 256):
                for j in range(2):
                    r = k * (_BGT // 128) + q * 2 + j
                    pltpu.sync_copy(shared.at[nidx.at[r]],
                                    buf.at[pl.ds(j * 128, 128)])
                pltpu.sync_copy(buf, g.at[k, pl.ds(gbase + q * 256, 256),
                                          pl.ds(0, CP)])

    @pl.when(cid == 0)
    def _():
        gather(n1, g1)

    @pl.when(cid == 1)
    def _():
        gather(n2, g2)


def _stage_splat_blur(w1rows, i1, w2rows, i2, nbr1, nbr2):
    zrows = jnp.zeros((_SPC, CP), jnp.float32)
    f = pl.kernel(
        _splat_blur_body,
        out_type=[jax.ShapeDtypeStruct((9, H, 128), jnp.float32)] * 2,
        mesh=plsc.VectorSubcoreMesh(**_MESH),
        compiler_params=pltpu.CompilerParams(use_tc_tiling_on_sc=False),
        scratch_types=[pltpu.VMEM_SHARED((H, CP), jnp.float32),
                       pltpu.VMEM((_SPC, CP), jnp.float32),
                       pltpu.VMEM((32, 128), jnp.int32),
                       pltpu.VMEM((9 * _BGT // 128, 128), jnp.int32)],
    )
    return f(w1rows, i1, w2rows, i2, zrows, nbr1, nbr2)


_BH3 = 512


def _blur_conv_body(g1, g2, wbk, bbr, o1, o2):
    for g, o in ((g1, o1), (g2, o2)):
        s = jnp.zeros((_BH3, 64), jnp.float32)
        for k in range(9):
            s = s + _dot(g[k][:, :CP], wbk[k])
        o[...] = _leaky(s + bbr[...])


def _stage_blur_conv(g1, g2, wbk, bbr):
    gs = pl.BlockSpec((9, _BH3, 128), lambda i: (0, i, 0))
    return pl.pallas_call(
        _blur_conv_body,
        grid=(H // _BH3,),
        in_specs=[gs, gs, pl.BlockSpec((9, CP, 64), lambda i: (0, 0, 0)),
                  pl.BlockSpec((1, 64), lambda i: (0, 0))],
        out_specs=[pl.BlockSpec((_BH3, 64), lambda i: (i, 0))] * 2,
        out_shape=[jax.ShapeDtypeStruct((H, 64), jnp.float32)] * 2,
    )(g1, g2, wbk, bbr)


_CGC = H // NW


def _corr_gather_body(l1, c1, l2, c2, p, bufA0, bufB0, bufA1, bufB1, idx1,
                      idx2, sem0, sem1):
    cid = lax.axis_index("c")
    sid = lax.axis_index("s")
    wid = cid * NS + sid
    base = pl.multiple_of(wid * _CGC, _CGC)
    pltpu.sync_copy(c1.at[wid], idx1)
    pltpu.sync_copy(c2.at[wid], idx2)
    bufs = ((bufA0, bufB0, sem0), (bufA1, bufB1, sem1))
    nch = 27 * (_CGC // 256)
    pend = [None, None]

    def fire(t):
        bA, bB, sem = bufs[t % 2]
        ds = []
        for j in range(2):
            r = (t // 2) * (_CGC // 128) + (t % 2) * 2 + j
            ds.append(pltpu.async_copy(l1.at[idx1.at[r]],
                                       bA.at[pl.ds(j * 128, 128)], sem))
            ds.append(pltpu.async_copy(l2.at[idx2.at[r]],
                                       bB.at[pl.ds(j * 128, 128)], sem))
        pend[t % 2] = ds

    fire(0)
    for t in range(nch):
        if t + 1 < nch:
            fire(t + 1)
        bA, bB, _ = bufs[t % 2]
        for d in pend[t % 2]:
            d.wait()

        def mul_row(r, carry):
            for cc in range(4):
                sl = pl.ds(cc * 16, 16)
                bA[r, sl] = bA[r, sl] * bB[r, sl]
            return carry

        lax.fori_loop(0, 256, mul_row, 0, unroll=4)
        pltpu.sync_copy(bA, p.at[t // 2, pl.ds(base + (t % 2) * 256, 256),
                                 pl.ds(0, 64)])


def _stage_corr_gather(lat1, co1, lat2, co2):
    f = pl.kernel(
        _corr_gather_body,
        out_type=jax.ShapeDtypeStruct((27, H, 128), jnp.float32),
        mesh=plsc.VectorSubcoreMesh(**_MESH),
        compiler_params=pltpu.CompilerParams(use_tc_tiling_on_sc=False),
        scratch_types=[pltpu.VMEM((256, 64), jnp.float32),
                       pltpu.VMEM((256, 64), jnp.float32),
                       pltpu.VMEM((256, 64), jnp.float32),
                       pltpu.VMEM((256, 64), jnp.float32),
                       pltpu.VMEM((112, 128), jnp.int32),
                       pltpu.VMEM((112, 128), jnp.int32),
                       pltpu.SemaphoreType.DMA,
                       pltpu.SemaphoreType.DMA],
    )
    return f(lat1, co1, lat2, co2)


_BH4 = 512


def _corr_conv_body(p, wc1t, bc1r, wc2t, bc2r, o):
    s = jnp.zeros((_BH4, 32), jnp.float32)
    for k in range(27):
        s = s + _leaky(_dot(p[k][:, :64], wc1t[...]) + bc1r[...])
    m = s * (1.0 / 27.0)
    o[...] = _leaky(_dot(m, wc2t[...]) + bc2r[...])


def _stage_corr_conv(p, wc1t, bc1r, wc2t, bc2r):
    return pl.pallas_call(
        _corr_conv_body,
        grid=(H // _BH4,),
        in_specs=[pl.BlockSpec((27, _BH4, 128), lambda i: (0, i, 0)),
                  pl.BlockSpec((64, 32), lambda i: (0, 0)),
                  pl.BlockSpec((1, 32), lambda i: (0, 0)),
                  pl.BlockSpec((32, 32), lambda i: (0, 0)),
                  pl.BlockSpec((1, 32), lambda i: (0, 0))],
        out_specs=pl.BlockSpec((_BH4, 32), lambda i: (i, 0)),
        out_shape=jax.ShapeDtypeStruct((H, 32), jnp.float32),
    )(p, wc1t, bc1r, wc2t, bc2r)


_SLC = N // NW


def _slice_gather_body(hlat, lat1, offf, gh, gu, bufh, bufu, idxbuf):
    cid = lax.axis_index("c")
    sid = lax.axis_index("s")
    wid = cid * NS + sid
    base = pl.multiple_of(wid * _SLC, _SLC)
    pltpu.sync_copy(offf.at[wid], idxbuf)
    for d in range(4):
        for q in range(_SLC // 256):
            for j in range(2):
                r = d * (_SLC // 128) + q * 2 + j
                pltpu.sync_copy(hlat.at[idxbuf.at[r]],
                                bufh.at[pl.ds(j * 128, 128)])
                pltpu.sync_copy(lat1.at[idxbuf.at[r]],
                                bufu.at[pl.ds(j * 128, 128)])
            pltpu.sync_copy(bufh, gh.at[d, pl.ds(base + q * 256, 256),
                                        pl.ds(0, 32)])
            pltpu.sync_copy(bufu, gu.at[d, pl.ds(base + q * 256, 256),
                                        pl.ds(0, 64)])


def _stage_slice_gather(hlat, lat1, offf):
    f = pl.kernel(
        _slice_gather_body,
        out_type=[jax.ShapeDtypeStruct((4, N, 128), jnp.float32),
                  jax.ShapeDtypeStruct((4, N, 128), jnp.float32)],
        mesh=plsc.VectorSubcoreMesh(**_MESH),
        compiler_params=pltpu.CompilerParams(use_tc_tiling_on_sc=False),
        scratch_types=[pltpu.VMEM((256, 32), jnp.float32),
                       pltpu.VMEM((256, 64), jnp.float32),
                       pltpu.VMEM((4 * _SLC // 128, 128), jnp.int32)],
    )
    return f(hlat, lat1, offf)


_BN5 = 512


def _final_body(gh, gu, baT, elT, wr1e, wr1c, br1r, wr2t, br2r, wr3t, br3r,
                wf2u, wf2r, bf2r, wf3t, bf3r, wf4t, bf4r, o):
    ba = baT[...]
    corr = jnp.zeros((_BN5, 32), jnp.float32)
    up = jnp.zeros((_BN5, 64), jnp.float32)
    for d in range(4):
        w = ba[:, d:d + 1]
        corr = corr + gh[d][:, :32] * w
        up = up + gu[d][:, :64] * w
    bf = jnp.bfloat16
    r = _leaky(_dot(elT[...], wr1e[...]) + _dot(corr, wr1c[...]) + br1r[...])
    r = _leaky(_dot(r, wr2t[...]) + br2r[...])
    r = _leaky(_dot(r, wr3t[...]) + br3r[...])
    x = _leaky(_dot(up.astype(bf), wf2u[...].astype(bf))
               + _dot(r.astype(bf), wf2r[...].astype(bf)) + bf2r[...])
    x = _leaky(_dot(x.astype(bf), wf3t[...].astype(bf)) + bf3r[...])
    o[...] = _dot(x.astype(bf), wf4t[...].astype(bf)) + bf4r[...]


def _stage_final(gh, gu, baT, elT, weights):
    bs = lambda *s: pl.BlockSpec(s, lambda i: (0,) * len(s))
    return pl.pallas_call(
        _final_body,
        grid=(N // _BN5,),
        in_specs=[pl.BlockSpec((4, _BN5, 128), lambda i: (0, i, 0)),
                  pl.BlockSpec((4, _BN5, 128), lambda i: (0, i, 0)),
                  pl.BlockSpec((_BN5, 4), lambda i: (i, 0)),
                  pl.BlockSpec((_BN5, 4), lambda i: (i, 0)),
                  bs(4, 64), bs(32, 64), bs(1, 64),
                  bs(64, 64), bs(1, 64), bs(64, 64), bs(1, 64),
                  bs(64, 1024), bs(64, 1024), bs(1, 1024),
                  bs(1024, 512), bs(1, 512), bs(512, 128), bs(1, 128)],
        out_specs=pl.BlockSpec((_BN5, 128), lambda i: (i, 0)),
        out_shape=jax.ShapeDtypeStruct((N, 128), jnp.float32),
    )(gh, gu, baT, elT, *weights)


def kernel(pc1, pc2, pc1_el_minus_gr, pc2_el_minus_gr, pc1_barycentric,
           pc2_barycentric, pc1_lattice_offset, pc2_lattice_offset,
           pc1_blur_neighbors, pc2_blur_neighbors, pc1_corr_indices,
           pc2_corr_indices, W1, b1, W2, b2, W3, b3, Wb, bb, Wc1, bc1, Wc2,
           bc2, Wr1, br1, Wr2, br2, Wr3, br3, Wf2, bf2, Wf3, bf3, Wf4, bf4):
    i32 = jnp.int32
    pc1T, pc2T = pc1[0].T, pc2[0].T
    el1T, el2T = pc1_el_minus_gr[0].T, pc2_el_minus_gr[0].T
    ba1T, ba2T = pc1_barycentric[0].T, pc2_barycentric[0].T
    def wmaj(a, k, c):
        r = a.astype(i32).reshape(k, NW, c).transpose(1, 0, 2)
        r = r.reshape(NW, k * c // 128, 128)
        pad = (-r.shape[1]) % 8
        return jnp.pad(r, ((0, 0), (0, pad), (0, 0)))
    idx1 = pc1_lattice_offset[0].astype(i32).reshape(NS, _RPT // 128, 128)
    idx2 = pc2_lattice_offset[0].astype(i32).reshape(NS, _RPT // 128, 128)
    tmaj = lambda a, k, c: (a.astype(i32).reshape(k, NS, c)
                            .transpose(1, 0, 2).reshape(NS, k * c // 128, 128))
    nbr1 = tmaj(pc1_blur_neighbors[0], 9, _BGT)
    nbr2 = tmaj(pc2_blur_neighbors[0], 9, _BGT)
    co1 = wmaj(pc1_corr_indices[0], 27, _CGC)
    co2 = wmaj(pc2_corr_indices[0], 27, _CGC)
    off1 = wmaj(pc1_lattice_offset[0], 4, _SLC)

    r1 = lambda b: b.reshape(1, -1)
    wbk = jnp.pad(jnp.transpose(Wb.reshape(64, 68, 9), (2, 1, 0)),
                  ((0, 0), (0, 12), (0, 0)))
    wf4t = jnp.pad(Wf4.T, ((0, 0), (0, 125)))
    bf4p = jnp.pad(bf4, (0, 125)).reshape(1, 128)

    w1r, w2r = _stage_points(pc1T, el1T, ba1T, pc2T, el2T, ba2T,
                             W1.T, r1(b1), W2.T, r1(b2), W3.T, r1(b3))
    g1, g2 = _stage_splat_blur(w1r.reshape(4 * N, 128), idx1,
                               w2r.reshape(4 * N, 128), idx2, nbr1, nbr2)
    lat1, lat2 = _stage_blur_conv(g1, g2, wbk, r1(bb))
    p = _stage_corr_gather(lat1, co1, lat2, co2)
    hlat = _stage_corr_conv(p, Wc1.T, r1(bc1), Wc2.T, r1(bc2))
    gh, gu = _stage_slice_gather(hlat, lat1, off1)
    weights = (Wr1[:, :4].T, Wr1[:, 4:].T, r1(br1), Wr2.T, r1(br2), Wr3.T,
               r1(br3), Wf2[:, :64].T, Wf2[:, 64:].T, r1(bf2), Wf3.T,
               r1(bf3), wf4t, bf4p)
    o = _stage_final(gh, gu, ba1T, el1T, weights)
    return o[:, :3].T[None]


# speedup vs baseline: 1.0758x; 1.0360x over previous
import jax
import jax.numpy as jnp
from jax import lax
from jax.experimental import pallas as pl
from jax.experimental.pallas import tpu as pltpu
from jax.experimental.pallas import tpu_sc as plsc

N = 32768
H = 16384
CP = 80
NC, NS = 2, 16
NW = NC * NS

_MESH = dict(core_axis_name="c", subcore_axis_name="s", num_cores=NC,
             num_subcores=NS)


def _leaky(x):
    return jnp.where(x >= 0, x, 0.1 * x)


def _dot(a, b):
    return jnp.dot(a, b, preferred_element_type=jnp.float32)


_BN1 = 1024


def _points_body(pc1T, el1T, ba1T, pc2T, el2T, ba2T, W1T, b1, W2T, b2, W3T,
                 b3, o1, o2):
    zpad = jnp.zeros((_BN1, 60), jnp.float32)
    for pcT, elT, baT, o in ((pc1T, el1T, ba1T, o1), (pc2T, el2T, ba2T, o2)):
        h = _leaky(_dot(pcT[...], W1T[...]) + b1[...])
        h = _leaky(_dot(h, W2T[...]) + b2[...])
        f = _leaky(_dot(h, W3T[...]) + b3[...])
        feat = jnp.concatenate([elT[...], f, zpad], axis=1)
        ba = baT[...]
        for d in range(4):
            o[d] = feat * ba[:, d:d + 1]


def _stage_points(pc1T, el1T, ba1T, pc2T, el2T, ba2T, W1T, b1r, W2T, b2r,
                  W3T, b3r):
    bn = lambda c: pl.BlockSpec((_BN1, c), lambda i: (i, 0))
    wf = lambda s: pl.BlockSpec(s, lambda i: (0,) * len(s))
    return pl.pallas_call(
        _points_body,
        grid=(N // _BN1,),
        in_specs=[bn(3), bn(4), bn(4), bn(3), bn(4), bn(4),
                  wf((3, 32)), wf((1, 32)), wf((32, 32)), wf((1, 32)),
                  wf((32, 64)), wf((1, 64))],
        out_specs=[pl.BlockSpec((4, _BN1, 128), lambda i: (0, i, 0))] * 2,
        out_shape=[jax.ShapeDtypeStruct((4, N, 128), jnp.float32)] * 2,
    )(pc1T, el1T, ba1T, pc2T, el2T, ba2T, W1T, b1r, W2T, b2r, W3T, b3r)


_SPC = 256
_RPT = 4 * N // NS


_BGT = H // NS


def _splat_blur_body(w1, i1, w2, i2, zrows, n1, n2, g1, g2, shared, buf,
                     sidx, nidx):
    cid = lax.axis_index("c")
    sid = lax.axis_index("s")
    zsl = H // NS
    zoff = pl.multiple_of(sid * zsl, zsl)
    for z in range(zsl // _SPC):
        pltpu.sync_copy(zrows, buf)
        pltpu.sync_copy(buf, shared.at[pl.ds(zoff + z * _SPC, _SPC)])
    plsc.subcore_barrier()

    def scatter(whbm, ihbm):
        base = sid * _RPT
        for c in range(_RPT // _SPC):
            if c % 16 == 0:
                pltpu.sync_copy(ihbm.at[sid, pl.ds((c // 16) * 32, 32)], sidx)
            off = pl.multiple_of(base + c * _SPC, _SPC)
            pltpu.sync_copy(whbm.at[pl.ds(off, _SPC), pl.ds(0, CP)], buf)
            for j in range(_SPC // 128):
                pltpu.sync_copy(buf.at[pl.ds(j * 128, 128)],
                                shared.at[sidx.at[2 * (c % 16) + j]],
                                add=True)

    @pl.when(cid == 0)
    def _():
        scatter(w1, i1)

    @pl.when(cid == 1)
    def _():
        scatter(w2, i2)

    plsc.subcore_barrier()

    def gather(nf, g):
        pltpu.sync_copy(nf.at[sid], nidx)
        gbase = pl.multiple_of(sid * _BGT, _BGT)
        for k in range(9):
            for q in range(_BGT // 256):
                for j in range(2):
                    r = k * (_BGT // 128) + q * 2 + j
                    pltpu.sync_copy(shared.at[nidx.at[r]],
                                    buf.at[pl.ds(j * 128, 128)])
                pltpu.sync_copy(buf, g.at[k, pl.ds(gbase + q * 256, 256),
                                          pl.ds(0, CP)])

    @pl.when(cid == 0)
    def _():
        gather(n1, g1)

    @pl.when(cid == 1)
    def _():
        gather(n2, g2)


def _stage_splat_blur(w1rows, i1, w2rows, i2, nbr1, nbr2):
    zrows = jnp.zeros((_SPC, CP), jnp.float32)
    f = pl.kernel(
        _splat_blur_body,
        out_type=[jax.ShapeDtypeStruct((9, H, 128), jnp.float32)] * 2,
        mesh=plsc.VectorSubcoreMesh(**_MESH),
        compiler_params=pltpu.CompilerParams(use_tc_tiling_on_sc=False),
        scratch_types=[pltpu.VMEM_SHARED((H, CP), jnp.float32),
                       pltpu.VMEM((_SPC, CP), jnp.float32),
                       pltpu.VMEM((32, 128), jnp.int32),
                       pltpu.VMEM((9 * _BGT // 128, 128), jnp.int32)],
    )
    return f(w1rows, i1, w2rows, i2, zrows, nbr1, nbr2)


_BH3 = 512


def _blur_conv_body(g1, g2, wbk, bbr, o1, o2):
    for g, o in ((g1, o1), (g2, o2)):
        s = jnp.zeros((_BH3, 64), jnp.float32)
        for k in range(9):
            s = s + _dot(g[k][:, :CP], wbk[k])
        o[...] = _leaky(s + bbr[...])


def _stage_blur_conv(g1, g2, wbk, bbr):
    gs = pl.BlockSpec((9, _BH3, 128), lambda i: (0, i, 0))
    return pl.pallas_call(
        _blur_conv_body,
        grid=(H // _BH3,),
        in_specs=[gs, gs, pl.BlockSpec((9, CP, 64), lambda i: (0, 0, 0)),
                  pl.BlockSpec((1, 64), lambda i: (0, 0))],
        out_specs=[pl.BlockSpec((_BH3, 64), lambda i: (i, 0))] * 2,
        out_shape=[jax.ShapeDtypeStruct((H, 64), jnp.float32)] * 2,
    )(g1, g2, wbk, bbr)


_CGC = H // NW


def _corr_gather_body(l1, c1, l2, c2, p, bufA0, bufB0, bufA1, bufB1, idx1,
                      idx2, sem0, sem1):
    cid = lax.axis_index("c")
    sid = lax.axis_index("s")
    wid = cid * NS + sid
    base = pl.multiple_of(wid * _CGC, _CGC)
    pltpu.sync_copy(c1.at[wid], idx1)
    pltpu.sync_copy(c2.at[wid], idx2)
    bufs = ((bufA0, bufB0, sem0), (bufA1, bufB1, sem1))
    nch = 27 * (_CGC // 256)
    pend = [None, None]

    def fire(t):
        bA, bB, sem = bufs[t % 2]
        ds = []
        for j in range(2):
            r = (t // 2) * (_CGC // 128) + (t % 2) * 2 + j
            ds.append(pltpu.async_copy(l1.at[idx1.at[r]],
                                       bA.at[pl.ds(j * 128, 128)], sem))
            ds.append(pltpu.async_copy(l2.at[idx2.at[r]],
                                       bB.at[pl.ds(j * 128, 128)], sem))
        pend[t % 2] = ds

    fire(0)
    for t in range(nch):
        if t + 1 < nch:
            fire(t + 1)
        bA, bB, _ = bufs[t % 2]
        for d in pend[t % 2]:
            d.wait()

        def mul_row(r, carry):
            for cc in range(4):
                sl = pl.ds(cc * 16, 16)
                bA[r, sl] = bA[r, sl] * bB[r, sl]
            return carry

        lax.fori_loop(0, 256, mul_row, 0, unroll=4)
        pltpu.sync_copy(bA, p.at[t // 2, pl.ds(base + (t % 2) * 256, 256),
                                 pl.ds(0, 64)])


def _stage_corr_gather(lat1, co1, lat2, co2):
    f = pl.kernel(
        _corr_gather_body,
        out_type=jax.ShapeDtypeStruct((27, H, 128), jnp.float32),
        mesh=plsc.VectorSubcoreMesh(**_MESH),
        compiler_params=pltpu.CompilerParams(use_tc_tiling_on_sc=False),
        scratch_types=[pltpu.VMEM((256, 64), jnp.float32),
                       pltpu.VMEM((256, 64), jnp.float32),
                       pltpu.VMEM((256, 64), jnp.float32),
                       pltpu.VMEM((256, 64), jnp.float32),
                       pltpu.VMEM((112, 128), jnp.int32),
                       pltpu.VMEM((112, 128), jnp.int32),
                       pltpu.SemaphoreType.DMA,
                       pltpu.SemaphoreType.DMA],
    )
    return f(lat1, co1, lat2, co2)


_BH4 = 512


def _corr_conv_body(p, wc1t, bc1r, wc2t, bc2r, o):
    s = jnp.zeros((_BH4, 32), jnp.float32)
    for k in range(27):
        s = s + _leaky(_dot(p[k][:, :64], wc1t[...]) + bc1r[...])
    m = s * (1.0 / 27.0)
    o[...] = _leaky(_dot(m, wc2t[...]) + bc2r[...])


def _stage_corr_conv(p, wc1t, bc1r, wc2t, bc2r):
    return pl.pallas_call(
        _corr_conv_body,
        grid=(H // _BH4,),
        in_specs=[pl.BlockSpec((27, _BH4, 128), lambda i: (0, i, 0)),
                  pl.BlockSpec((64, 32), lambda i: (0, 0)),
                  pl.BlockSpec((1, 32), lambda i: (0, 0)),
                  pl.BlockSpec((32, 32), lambda i: (0, 0)),
                  pl.BlockSpec((1, 32), lambda i: (0, 0))],
        out_specs=pl.BlockSpec((_BH4, 32), lambda i: (i, 0)),
        out_shape=jax.ShapeDtypeStruct((H, 32), jnp.float32),
    )(p, wc1t, bc1r, wc2t, bc2r)


_SLC = N // NW


def _slice_gather_body(hlat, lat1, offf, gh, gu, bufh, bufu, bufh1, bufu1,
                       idxbuf, sem0, sem1):
    cid = lax.axis_index("c")
    sid = lax.axis_index("s")
    wid = cid * NS + sid
    base = pl.multiple_of(wid * _SLC, _SLC)
    pltpu.sync_copy(offf.at[wid], idxbuf)
    bufs = ((bufh, bufu, sem0), (bufh1, bufu1, sem1))
    pend = [None, None]
    nch = 4 * (_SLC // 256)

    def fire(t):
        bh, bu, sem = bufs[t % 2]
        ds = []
        for j in range(2):
            r = (t // 4) * (_SLC // 128) + (t % 4) * 2 + j
            ds.append(pltpu.async_copy(hlat.at[idxbuf.at[r]],
                                       bh.at[pl.ds(j * 128, 128)], sem))
            ds.append(pltpu.async_copy(lat1.at[idxbuf.at[r]],
                                       bu.at[pl.ds(j * 128, 128)], sem))
        pend[t % 2] = ds

    fire(0)
    for t in range(nch):
        if t + 1 < nch:
            fire(t + 1)
        bh, bu, _ = bufs[t % 2]
        for d in pend[t % 2]:
            d.wait()
        row = pl.ds(base + (t % 4) * 256, 256)
        pltpu.sync_copy(bh, gh.at[t // 4, row, pl.ds(0, 32)])
        pltpu.sync_copy(bu, gu.at[t // 4, row, pl.ds(0, 64)])


def _stage_slice_gather(hlat, lat1, offf):
    f = pl.kernel(
        _slice_gather_body,
        out_type=[jax.ShapeDtypeStruct((4, N, 128), jnp.float32),
                  jax.ShapeDtypeStruct((4, N, 128), jnp.float32)],
        mesh=plsc.VectorSubcoreMesh(**_MESH),
        compiler_params=pltpu.CompilerParams(use_tc_tiling_on_sc=False),
        scratch_types=[pltpu.VMEM((256, 32), jnp.float32),
                       pltpu.VMEM((256, 64), jnp.float32),
                       pltpu.VMEM((256, 32), jnp.float32),
                       pltpu.VMEM((256, 64), jnp.float32),
                       pltpu.VMEM((4 * _SLC // 128, 128), jnp.int32),
                       pltpu.SemaphoreType.DMA,
                       pltpu.SemaphoreType.DMA],
    )
    return f(hlat, lat1, offf)


_BN5 = 512


def _final_body(gh, gu, baT, elT, wr1e, wr1c, br1r, wr2t, br2r, wr3t, br3r,
                wf2u, wf2r, bf2r, wf3t, bf3r, wf4t, bf4r, o):
    ba = baT[...]
    corr = jnp.zeros((_BN5, 32), jnp.float32)
    up = jnp.zeros((_BN5, 64), jnp.float32)
    for d in range(4):
        w = ba[:, d:d + 1]
        corr = corr + gh[d][:, :32] * w
        up = up + gu[d][:, :64] * w
    bf = jnp.bfloat16
    r = _leaky(_dot(elT[...], wr1e[...]) + _dot(corr, wr1c[...]) + br1r[...])
    r = _leaky(_dot(r, wr2t[...]) + br2r[...])
    r = _leaky(_dot(r, wr3t[...]) + br3r[...])
    x = _leaky(_dot(up.astype(bf), wf2u[...].astype(bf))
               + _dot(r.astype(bf), wf2r[...].astype(bf)) + bf2r[...])
    x = _leaky(_dot(x.astype(bf), wf3t[...].astype(bf)) + bf3r[...])
    o[...] = _dot(x.astype(bf), wf4t[...].astype(bf)) + bf4r[...]


def _stage_final(gh, gu, baT, elT, weights):
    bs = lambda *s: pl.BlockSpec(s, lambda i: (0,) * len(s))
    return pl.pallas_call(
        _final_body,
        grid=(N // _BN5,),
        in_specs=[pl.BlockSpec((4, _BN5, 128), lambda i: (0, i, 0)),
                  pl.BlockSpec((4, _BN5, 128), lambda i: (0, i, 0)),
                  pl.BlockSpec((_BN5, 4), lambda i: (i, 0)),
                  pl.BlockSpec((_BN5, 4), lambda i: (i, 0)),
                  bs(4, 64), bs(32, 64), bs(1, 64),
                  bs(64, 64), bs(1, 64), bs(64, 64), bs(1, 64),
                  bs(64, 1024), bs(64, 1024), bs(1, 1024),
                  bs(1024, 512), bs(1, 512), bs(512, 128), bs(1, 128)],
        out_specs=pl.BlockSpec((_BN5, 128), lambda i: (i, 0)),
        out_shape=jax.ShapeDtypeStruct((N, 128), jnp.float32),
    )(gh, gu, baT, elT, *weights)


def kernel(pc1, pc2, pc1_el_minus_gr, pc2_el_minus_gr, pc1_barycentric,
           pc2_barycentric, pc1_lattice_offset, pc2_lattice_offset,
           pc1_blur_neighbors, pc2_blur_neighbors, pc1_corr_indices,
           pc2_corr_indices, W1, b1, W2, b2, W3, b3, Wb, bb, Wc1, bc1, Wc2,
           bc2, Wr1, br1, Wr2, br2, Wr3, br3, Wf2, bf2, Wf3, bf3, Wf4, bf4):
    i32 = jnp.int32
    pc1T, pc2T = pc1[0].T, pc2[0].T
    el1T, el2T = pc1_el_minus_gr[0].T, pc2_el_minus_gr[0].T
    ba1T, ba2T = pc1_barycentric[0].T, pc2_barycentric[0].T
    def wmaj(a, k, c):
        r = a.astype(i32).reshape(k, NW, c).transpose(1, 0, 2)
        r = r.reshape(NW, k * c // 128, 128)
        pad = (-r.shape[1]) % 8
        return jnp.pad(r, ((0, 0), (0, pad), (0, 0)))
    idx1 = pc1_lattice_offset[0].astype(i32).reshape(NS, _RPT // 128, 128)
    idx2 = pc2_lattice_offset[0].astype(i32).reshape(NS, _RPT // 128, 128)
    tmaj = lambda a, k, c: (a.astype(i32).reshape(k, NS, c)
                            .transpose(1, 0, 2).reshape(NS, k * c // 128, 128))
    nbr1 = tmaj(pc1_blur_neighbors[0], 9, _BGT)
    nbr2 = tmaj(pc2_blur_neighbors[0], 9, _BGT)
    co1 = wmaj(pc1_corr_indices[0], 27, _CGC)
    co2 = wmaj(pc2_corr_indices[0], 27, _CGC)
    off1 = wmaj(pc1_lattice_offset[0], 4, _SLC)

    r1 = lambda b: b.reshape(1, -1)
    wbk = jnp.pad(jnp.transpose(Wb.reshape(64, 68, 9), (2, 1, 0)),
                  ((0, 0), (0, 12), (0, 0)))
    wf4t = jnp.pad(Wf4.T, ((0, 0), (0, 125)))
    bf4p = jnp.pad(bf4, (0, 125)).reshape(1, 128)

    w1r, w2r = _stage_points(pc1T, el1T, ba1T, pc2T, el2T, ba2T,
                             W1.T, r1(b1), W2.T, r1(b2), W3.T, r1(b3))
    g1, g2 = _stage_splat_blur(w1r.reshape(4 * N, 128), idx1,
                               w2r.reshape(4 * N, 128), idx2, nbr1, nbr2)
    lat1, lat2 = _stage_blur_conv(g1, g2, wbk, r1(bb))
    p = _stage_corr_gather(lat1, co1, lat2, co2)
    hlat = _stage_corr_conv(p, Wc1.T, r1(bc1), Wc2.T, r1(bc2))
    gh, gu = _stage_slice_gather(hlat, lat1, off1)
    weights = (Wr1[:, :4].T, Wr1[:, 4:].T, r1(br1), Wr2.T, r1(br2), Wr3.T,
               r1(br3), Wf2[:, :64].T, Wf2[:, 64:].T, r1(bf2), Wf3.T,
               r1(bf3), wf4t, bf4p)
    o = _stage_final(gh, gu, ba1T, el1T, weights)
    return o[:, :3].T[None]
